# baseline (speedup 1.0000x reference)
import dataclasses
import functools

import jax
import jax.numpy as jnp
from jax import lax
from jax.experimental import pallas as pl
from jax.experimental.pallas import tpu as pltpu
from jax.experimental.pallas import tpu_sc as plsc

N = 10000
N_AL = 10240
E = 320000
D = 128
R = 4
M = 1024

NC = 2
NS = 16
NW = NC * NS

CE = 128
NCH = E // CE
CP = 80
NPCH = N // CP

HW = 128
f32 = jnp.float32
i32 = jnp.int32


def _sc_idx(src, dst, et, hier):
    @functools.partial(
        pl.kernel,
        out_type=[
            jax.ShapeDtypeStruct((E,), i32),
            jax.ShapeDtypeStruct((E,), i32),
            jax.ShapeDtypeStruct((E,), i32),
        ],
        mesh=_mesh(),
        compiler_params=_sc_params(),
        scratch_types=[
            pltpu.VMEM((N,), i32),
            pltpu.VMEM((CE,), i32),
            pltpu.VMEM((CE,), i32),
            pltpu.VMEM((CE,), i32),
            pltpu.VMEM((CE,), i32),
            pltpu.VMEM((CE,), i32),
            pltpu.VMEM((CE,), i32),
        ],
    )
    def k(src_h, dst_h, et_h, hier_h, ri_out, mri_out, md_out,
          hier_v, s_v, d_v, t_v, ri_v, mri_v, md_v):
        cid = lax.axis_index("c")
        sid = lax.axis_index("s")
        wid = cid * NS + sid
        pltpu.sync_copy(hier_h, hier_v)

        @pl.loop(wid, NCH, step=NW)
        def _(c):
            base = c * CE
            pltpu.sync_copy(src_h.at[pl.ds(base, CE)], s_v)
            pltpu.sync_copy(dst_h.at[pl.ds(base, CE)], d_v)
            pltpu.sync_copy(et_h.at[pl.ds(base, CE)], t_v)
            for j in range(CE // 16):
                sl = pl.ds(j * 16, 16)
                s16 = s_v[sl]
                d16 = d_v[sl]
                t16 = t_v[sl]
                ri_v[sl] = t16 * N_AL + s16
                md_v[sl] = plsc.load_gather(hier_v, [d16])
                mri_v[sl] = t16 * M + plsc.load_gather(hier_v, [s16])
            pltpu.sync_copy(ri_v, ri_out.at[pl.ds(base, CE)])
            pltpu.sync_copy(mri_v, mri_out.at[pl.ds(base, CE)])
            pltpu.sync_copy(md_v, md_out.at[pl.ds(base, CE)])

    return k(src, dst, et, hier)


def _sc_hist(idx, BINS, LEN, CHUNK):
    NCHK = LEN // CHUNK

    @functools.partial(
        pl.kernel,
        out_type=jax.ShapeDtypeStruct((NC, NS, BINS), f32),
        mesh=_mesh(),
        compiler_params=_sc_params(),
        scratch_types=[
            pltpu.VMEM((CHUNK,), i32),
            pltpu.VMEM((BINS,), f32),
        ],
    )
    def k(idx_h, out_h, idx_v, hist_v):
        cid = lax.axis_index("c")
        sid = lax.axis_index("s")
        wid = cid * NS + sid
        z16 = jnp.zeros((16,), f32)
        ones16 = jnp.ones((16,), f32)

        @pl.loop(0, BINS // 16)
        def _(i):
            hist_v[pl.ds(i * 16, 16)] = z16

        @pl.loop(wid, NCHK, step=NW)
        def _(c):
            pltpu.sync_copy(idx_h.at[pl.ds(c * CHUNK, CHUNK)], idx_v)
            for j in range(CHUNK // 16):
                i16 = idx_v[pl.ds(j * 16, 16)]
                plsc.addupdate_scatter(hist_v, [i16], ones16)

        pltpu.sync_copy(hist_v, out_h.at[cid, sid])

    return k(idx)


def _mesh():
    return plsc.VectorSubcoreMesh(core_axis_name="c", subcore_axis_name="s")


def _sc_params():
    cp = pltpu.CompilerParams()
    if "needs_layout_passes" in pltpu.CompilerParams.__dataclass_fields__:
        cp = dataclasses.replace(cp, needs_layout_passes=False)
    return cp


CPT = NCH // NW
REM = NCH - CPT * NW


def _sc_edge_agg(table, ridx, dsti, NN, zerosD, G):
    NPT = NN // NS
    NGRP = CPT // G
    LEFT = CPT - NGRP * G

    @functools.partial(
        pl.kernel,
        out_type=jax.ShapeDtypeStruct((NC, NN, D), f32),
        mesh=_mesh(),
        scratch_types=(
            [pltpu.VMEM((G, CE), i32), pltpu.VMEM((G, CE), i32)]
            + [pltpu.VMEM((CE, D), f32)] * G
            + [pltpu.VMEM_SHARED((NN, D), f32)]
            + [pltpu.SemaphoreType.DMA] * G
        ),
    )
    def k(tab_h, ri_h, di_h, zeros_h, out_h, ri_v, di_v, *rest):
        rows = rest[:G]
        agg_s = rest[G]
        sems = rest[G + 1:]
        cid = lax.axis_index("c")
        sid = lax.axis_index("s")
        wid = cid * NS + sid
        r0 = rows[0]
        pltpu.sync_copy(zeros_h, r0)
        off = 0
        while off < NPT:
            sz = min(CE, NPT - off)
            pltpu.sync_copy(r0.at[pl.ds(0, sz)],
                            agg_s.at[pl.ds(sid * NPT + off, sz)])
            off += sz
        plsc.subcore_barrier()

        def do_chunks(cs):
            for j, c in enumerate(cs):
                pltpu.sync_copy(ri_h.at[pl.ds(c * CE, CE)], ri_v.at[j])
                pltpu.sync_copy(di_h.at[pl.ds(c * CE, CE)], di_v.at[j])
            copies = [
                pltpu.async_copy(tab_h.at[ri_v.at[j]], rows[j], sems[j])
                for j in range(len(cs))
            ]
            for j in range(len(cs)):
                copies[j].wait()
                pltpu.sync_copy(rows[j], agg_s.at[di_v.at[j]], add=True)

        @pl.loop(0, NGRP)
        def _(g):
            do_chunks([wid + (g * G + j) * NW for j in range(G)])

        if LEFT:
            do_chunks([wid + (NGRP * G + j) * NW for j in range(LEFT)])

        @pl.when(wid < REM)
        def _():
            do_chunks([wid + CPT * NW])

        plsc.subcore_barrier()
        off = 0
        while off < NPT:
            sz = min(CE, NPT - off)
            pltpu.sync_copy(agg_s.at[pl.ds(sid * NPT + off, sz)],
                            out_h.at[cid, pl.ds(sid * NPT + off, sz)])
            off += sz

    return k(table, ridx, dsti, zerosD)


def _sc_pool(h2, hier, zerosD):
    MPT = M // NS

    @functools.partial(
        pl.kernel,
        out_type=jax.ShapeDtypeStruct((NC, M, D), f32),
        mesh=_mesh(),
        scratch_types=[
            pltpu.VMEM((CP,), i32),
            pltpu.VMEM((CP, D), f32),
            pltpu.VMEM((CE, D), f32),
            pltpu.VMEM_SHARED((M, D), f32),
        ],
    )
    def k(h_h, hier_h, zeros_h, out_h, hidx_v, rows_v, zbuf_v, pool_s):
        cid = lax.axis_index("c")
        sid = lax.axis_index("s")
        wid = cid * NS + sid
        pltpu.sync_copy(zeros_h, zbuf_v)
        pltpu.sync_copy(zbuf_v.at[pl.ds(0, MPT)],
                        pool_s.at[pl.ds(sid * MPT, MPT)])
        plsc.subcore_barrier()

        @pl.loop(wid, NPCH, step=NW)
        def _(c):
            base = c * CP
            pltpu.sync_copy(h_h.at[pl.ds(base, CP)], rows_v)
            pltpu.sync_copy(hier_h.at[pl.ds(base, CP)], hidx_v)
            pltpu.sync_copy(rows_v, pool_s.at[hidx_v], add=True)

        plsc.subcore_barrier()
        pltpu.sync_copy(pool_s.at[pl.ds(sid * MPT, MPT)],
                        out_h.at[cid, pl.ds(sid * MPT, MPT)])

    return k(h2, hier, zerosD)


def _tc_first(x, Wi, bi, Wrel, Wself, b):
    BN = 512
    G = N_AL // BN

    def body(x_r, wi_r, bi_r, wr_r, ws_r, b_r, hr_r, st_r):
        h = jnp.maximum(
            jnp.dot(x_r[...], wi_r[...], preferred_element_type=f32) + bi_r[...],
            0.0)
        for r in range(R):
            hr_r[r] = jnp.dot(h, wr_r[r], preferred_element_type=f32)
        st_r[...] = jnp.dot(h, ws_r[...], preferred_element_type=f32) + b_r[...]

    return pl.pallas_call(
        body,
        grid=(G,),
        in_specs=[
            pl.BlockSpec((BN, D), lambda i: (i, 0)),
            pl.BlockSpec((D, D), lambda i: (0, 0)),
            pl.BlockSpec((1, D), lambda i: (0, 0)),
            pl.BlockSpec((R, D, D), lambda i: (0, 0, 0)),
            pl.BlockSpec((D, D), lambda i: (0, 0)),
            pl.BlockSpec((1, D), lambda i: (0, 0)),
        ],
        out_specs=[
            pl.BlockSpec((R, BN, D), lambda i: (0, i, 0)),
            pl.BlockSpec((BN, D), lambda i: (i, 0)),
        ],
        out_shape=[
            jax.ShapeDtypeStruct((R, N_AL, D), f32),
            jax.ShapeDtypeStruct((N_AL, D), f32),
        ],
    )(x, Wi, bi, Wrel, Wself, b)


def _tc_mid(aggP, degP, st, Wrel, Wself, b, NN, BN):
    G = NN // BN

    def body(ap_r, dp_r, st_r, wr_r, ws_r, b_r, hr_r, st_o):
        agg = ap_r[0] + ap_r[1]
        deg = jnp.maximum(jnp.sum(dp_r[...], axis=(0, 1)), 1.0)
        h = jnp.maximum(agg / deg[:, None] + st_r[...], 0.0)
        for r in range(R):
            hr_r[r] = jnp.dot(h, wr_r[r], preferred_element_type=f32)
        st_o[...] = jnp.dot(h, ws_r[...], preferred_element_type=f32) + b_r[...]

    return pl.pallas_call(
        body,
        grid=(G,),
        in_specs=[
            pl.BlockSpec((NC, BN, D), lambda i: (0, i, 0)),
            pl.BlockSpec((NC, NS, BN), lambda i: (0, 0, i)),
            pl.BlockSpec((BN, D), lambda i: (i, 0)),
            pl.BlockSpec((R, D, D), lambda i: (0, 0, 0)),
            pl.BlockSpec((D, D), lambda i: (0, 0)),
            pl.BlockSpec((1, D), lambda i: (0, 0)),
        ],
        out_specs=[
            pl.BlockSpec((R, BN, D), lambda i: (0, i, 0)),
            pl.BlockSpec((BN, D), lambda i: (i, 0)),
        ],
        out_shape=[
            jax.ShapeDtypeStruct((R, NN, D), f32),
            jax.ShapeDtypeStruct((NN, D), f32),
        ],
    )(aggP, degP, st, Wrel, Wself, b)


def _tc_h(aggP, degP, st, NN, BN):
    G = NN // BN

    def body(ap_r, dp_r, st_r, h_o):
        agg = ap_r[0] + ap_r[1]
        deg = jnp.maximum(jnp.sum(dp_r[...], axis=(0, 1)), 1.0)
        h_o[...] = jnp.maximum(agg / deg[:, None] + st_r[...], 0.0)

    return pl.pallas_call(
        body,
        grid=(G,),
        in_specs=[
            pl.BlockSpec((NC, BN, D), lambda i: (0, i, 0)),
            pl.BlockSpec((NC, NS, BN), lambda i: (0, 0, i)),
            pl.BlockSpec((BN, D), lambda i: (i, 0)),
        ],
        out_specs=pl.BlockSpec((BN, D), lambda i: (i, 0)),
        out_shape=jax.ShapeDtypeStruct((NN, D), f32),
    )(aggP, degP, st)


def _tc_pool_mlp(poolP, cntP, Wrel, Wself, b):
    BN = 256
    G = M // BN

    def body(pp_r, cp_r, wr_r, ws_r, b_r, pr_r, st_o):
        cnt = jnp.maximum(jnp.sum(cp_r[...], axis=(0, 1)), 1.0)
        pooled = (pp_r[0] + pp_r[1]) / cnt[:, None]
        for r in range(R):
            pr_r[r] = jnp.dot(pooled, wr_r[r], preferred_element_type=f32)
        st_o[...] = jnp.dot(pooled, ws_r[...], preferred_element_type=f32) + b_r[...]

    return pl.pallas_call(
        body,
        grid=(G,),
        in_specs=[
            pl.BlockSpec((NC, BN, D), lambda i: (0, i, 0)),
            pl.BlockSpec((NC, NS, BN), lambda i: (0, 0, i)),
            pl.BlockSpec((R, D, D), lambda i: (0, 0, 0)),
            pl.BlockSpec((D, D), lambda i: (0, 0)),
            pl.BlockSpec((1, D), lambda i: (0, 0)),
        ],
        out_specs=[
            pl.BlockSpec((R, BN, D), lambda i: (0, i, 0)),
            pl.BlockSpec((BN, D), lambda i: (i, 0)),
        ],
        out_shape=[
            jax.ShapeDtypeStruct((R, M, D), f32),
            jax.ShapeDtypeStruct((M, D), f32),
        ],
    )(poolP, cntP, Wrel, Wself, b)


def _tc_final(maggP, degmP, pst, Wf, bf):
    BN = 256
    G = M // BN

    def body(ap_r, dp_r, st_r, wf_r, bf_r, o_r):
        deg = jnp.maximum(jnp.sum(dp_r[...], axis=(0, 1)), 1.0)
        p2 = jnp.maximum((ap_r[0] + ap_r[1]) / deg[:, None] + st_r[...], 0.0)
        o_r[...] = jnp.maximum(
            jnp.dot(p2, wf_r[...], preferred_element_type=f32) + bf_r[...], 0.0)

    return pl.pallas_call(
        body,
        grid=(G,),
        in_specs=[
            pl.BlockSpec((NC, BN, D), lambda i: (0, i, 0)),
            pl.BlockSpec((NC, NS, BN), lambda i: (0, 0, i)),
            pl.BlockSpec((BN, D), lambda i: (i, 0)),
            pl.BlockSpec((D, D), lambda i: (0, 0)),
            pl.BlockSpec((1, D), lambda i: (0, 0)),
        ],
        out_specs=pl.BlockSpec((BN, D), lambda i: (i, 0)),
        out_shape=jax.ShapeDtypeStruct((M, D), f32),
    )(maggP, degmP, pst, Wf, bf)


def kernel(x, edge_index, edge_type, hierarchy,
           W_init, b_init, W_rel_bu, W_self_bu, b_bu,
           W_rel_mod, W_self_mod, b_mod, W_fin, b_fin):
    src = edge_index[0]
    dst = edge_index[1]
    zerosD = jnp.zeros((CE, D), f32)

    ridx, mridx, mdst = _sc_idx(src, dst, edge_type, hierarchy)

    def _tok_i32(v):
        return jnp.minimum(v.astype(i32), 0)

    degP = _sc_hist(dst + _tok_i32(ridx[0]), N_AL, E, CE)
    degmP = _sc_hist(mdst + _tok_i32(degP[0, 0, 0]), M, E, CE)
    cntP = _sc_hist(hierarchy + _tok_i32(degmP[0, 0, 0]), M, N, CP)
    ridx = ridx + _tok_i32(cntP[0, 0, 0])

    x_pad = jnp.concatenate([x, jnp.zeros((N_AL - N, D), f32)], axis=0)
    hr0, st0 = _tc_first(x_pad, W_init, b_init.reshape(1, D),
                         W_rel_bu[0], W_self_bu[0], b_bu[0].reshape(1, D))
    aggP0 = _sc_edge_agg(hr0.reshape(R * N_AL, D), ridx, dst, N_AL, zerosD, 2)
    hr1, st1 = _tc_mid(aggP0, degP, st0,
                       W_rel_bu[1], W_self_bu[1], b_bu[1].reshape(1, D),
                       N_AL, 512)
    aggP1 = _sc_edge_agg(hr1.reshape(R * N_AL, D), ridx, dst, N_AL, zerosD, 2)
    h2 = _tc_h(aggP1, degP, st1, N_AL, 512)

    poolP = _sc_pool(h2, hierarchy, zerosD)
    pr0, pst0 = _tc_pool_mlp(poolP, cntP,
                             W_rel_mod[0], W_self_mod[0], b_mod[0].reshape(1, D))
    maggP0 = _sc_edge_agg(pr0.reshape(R * M, D), mridx, mdst, M, zerosD, 4)
    pr1, pst1 = _tc_mid(maggP0, degmP, pst0,
                        W_rel_mod[1], W_self_mod[1], b_mod[1].reshape(1, D),
                        M, 256)
    maggP1 = _sc_edge_agg(pr1.reshape(R * M, D), mridx, mdst, M, zerosD, 4)
    return _tc_final(maggP1, degmP, pst1, W_fin, b_fin.reshape(1, D))


# speedup vs baseline: 1.0041x; 1.0041x over previous
import dataclasses
import functools

import jax
import jax.numpy as jnp
from jax import lax
from jax.experimental import pallas as pl
from jax.experimental.pallas import tpu as pltpu
from jax.experimental.pallas import tpu_sc as plsc

N = 10000
N_AL = 10240
E = 320000
D = 128
R = 4
M = 1024

NC = 2
NS = 16
NW = NC * NS

CE = 128
NCH = E // CE
CP = 80
NPCH = N // CP

f32 = jnp.float32
i32 = jnp.int32


def _sc_idx(src, dst, et, hier):
    @functools.partial(
        pl.kernel,
        out_type=[
            jax.ShapeDtypeStruct((E,), i32),
            jax.ShapeDtypeStruct((E,), i32),
            jax.ShapeDtypeStruct((E,), i32),
        ],
        mesh=_mesh(),
        compiler_params=_sc_params(),
        scratch_types=[
            pltpu.VMEM((N,), i32),
            pltpu.VMEM((CE,), i32),
            pltpu.VMEM((CE,), i32),
            pltpu.VMEM((CE,), i32),
            pltpu.VMEM((CE,), i32),
            pltpu.VMEM((CE,), i32),
            pltpu.VMEM((CE,), i32),
        ],
    )
    def k(src_h, dst_h, et_h, hier_h, ri_out, mri_out, md_out,
          hier_v, s_v, d_v, t_v, ri_v, mri_v, md_v):
        cid = lax.axis_index("c")
        sid = lax.axis_index("s")
        wid = cid * NS + sid
        pltpu.sync_copy(hier_h, hier_v)

        @pl.loop(wid, NCH, step=NW)
        def _(c):
            base = c * CE
            pltpu.sync_copy(src_h.at[pl.ds(base, CE)], s_v)
            pltpu.sync_copy(dst_h.at[pl.ds(base, CE)], d_v)
            pltpu.sync_copy(et_h.at[pl.ds(base, CE)], t_v)
            for j in range(CE // 16):
                sl = pl.ds(j * 16, 16)
                s16 = s_v[sl]
                d16 = d_v[sl]
                t16 = t_v[sl]
                ri_v[sl] = t16 * N_AL + s16
                md_v[sl] = plsc.load_gather(hier_v, [d16])
                mri_v[sl] = t16 * M + plsc.load_gather(hier_v, [s16])
            pltpu.sync_copy(ri_v, ri_out.at[pl.ds(base, CE)])
            pltpu.sync_copy(mri_v, mri_out.at[pl.ds(base, CE)])
            pltpu.sync_copy(md_v, md_out.at[pl.ds(base, CE)])

    return k(src, dst, et, hier)


def _sc_hist(idx, BINS, LEN, CHUNK):
    NCHK = LEN // CHUNK

    @functools.partial(
        pl.kernel,
        out_type=jax.ShapeDtypeStruct((NC, NS, BINS), f32),
        mesh=_mesh(),
        compiler_params=_sc_params(),
        scratch_types=[
            pltpu.VMEM((CHUNK,), i32),
            pltpu.VMEM((BINS,), f32),
        ],
    )
    def k(idx_h, out_h, idx_v, hist_v):
        cid = lax.axis_index("c")
        sid = lax.axis_index("s")
        wid = cid * NS + sid
        z16 = jnp.zeros((16,), f32)
        ones16 = jnp.ones((16,), f32)

        @pl.loop(0, BINS // 16)
        def _(i):
            hist_v[pl.ds(i * 16, 16)] = z16

        @pl.loop(wid, NCHK, step=NW)
        def _(c):
            pltpu.sync_copy(idx_h.at[pl.ds(c * CHUNK, CHUNK)], idx_v)
            for j in range(CHUNK // 16):
                i16 = idx_v[pl.ds(j * 16, 16)]
                plsc.addupdate_scatter(hist_v, [i16], ones16)

        pltpu.sync_copy(hist_v, out_h.at[cid, sid])

    return k(idx)


def _mesh():
    return plsc.VectorSubcoreMesh(core_axis_name="c", subcore_axis_name="s")


def _sc_params():
    cp = pltpu.CompilerParams()
    if "needs_layout_passes" in pltpu.CompilerParams.__dataclass_fields__:
        cp = dataclasses.replace(cp, needs_layout_passes=False)
    return cp


CPT = NCH // NW
REM = NCH - CPT * NW


def _sc_edge_agg(table, ridx, dsti, NN, zerosD, G):
    NPT = NN // NS
    NGRP = CPT // G
    LEFT = CPT - NGRP * G

    @functools.partial(
        pl.kernel,
        out_type=jax.ShapeDtypeStruct((NC, NN, D), f32),
        mesh=_mesh(),
        scratch_types=(
            [pltpu.VMEM((G, CE), i32), pltpu.VMEM((G, CE), i32)]
            + [pltpu.VMEM((CE, D), f32)] * G
            + [pltpu.VMEM_SHARED((NN, D), f32)]
            + [pltpu.SemaphoreType.DMA] * G
        ),
    )
    def k(tab_h, ri_h, di_h, zeros_h, out_h, ri_v, di_v, *rest):
        rows = rest[:G]
        agg_s = rest[G]
        sems = rest[G + 1:]
        cid = lax.axis_index("c")
        sid = lax.axis_index("s")
        wid = cid * NS + sid
        r0 = rows[0]
        pltpu.sync_copy(zeros_h, r0)
        off = 0
        while off < NPT:
            sz = min(CE, NPT - off)
            pltpu.sync_copy(r0.at[pl.ds(0, sz)],
                            agg_s.at[pl.ds(sid * NPT + off, sz)])
            off += sz
        plsc.subcore_barrier()

        def do_chunks(cs):
            for j, c in enumerate(cs):
                pltpu.sync_copy(ri_h.at[pl.ds(c * CE, CE)], ri_v.at[j])
                pltpu.sync_copy(di_h.at[pl.ds(c * CE, CE)], di_v.at[j])
            copies = [
                pltpu.async_copy(tab_h.at[ri_v.at[j]], rows[j], sems[j])
                for j in range(len(cs))
            ]
            for j in range(len(cs)):
                copies[j].wait()
                pltpu.sync_copy(rows[j], agg_s.at[di_v.at[j]], add=True)

        @pl.loop(0, NGRP)
        def _(g):
            do_chunks([wid + (g * G + j) * NW for j in range(G)])

        if LEFT:
            do_chunks([wid + (NGRP * G + j) * NW for j in range(LEFT)])

        @pl.when(wid < REM)
        def _():
            do_chunks([wid + CPT * NW])

        plsc.subcore_barrier()
        off = 0
        while off < NPT:
            sz = min(CE, NPT - off)
            pltpu.sync_copy(agg_s.at[pl.ds(sid * NPT + off, sz)],
                            out_h.at[cid, pl.ds(sid * NPT + off, sz)])
            off += sz

    return k(table, ridx, dsti, zerosD)


def _sc_pool(h2, hier, zerosD):
    MPT = M // NS

    @functools.partial(
        pl.kernel,
        out_type=jax.ShapeDtypeStruct((NC, M, D), f32),
        mesh=_mesh(),
        scratch_types=[
            pltpu.VMEM((CP,), i32),
            pltpu.VMEM((CP, D), f32),
            pltpu.VMEM((CE, D), f32),
            pltpu.VMEM_SHARED((M, D), f32),
        ],
    )
    def k(h_h, hier_h, zeros_h, out_h, hidx_v, rows_v, zbuf_v, pool_s):
        cid = lax.axis_index("c")
        sid = lax.axis_index("s")
        wid = cid * NS + sid
        pltpu.sync_copy(zeros_h, zbuf_v)
        pltpu.sync_copy(zbuf_v.at[pl.ds(0, MPT)],
                        pool_s.at[pl.ds(sid * MPT, MPT)])
        plsc.subcore_barrier()

        @pl.loop(wid, NPCH, step=NW)
        def _(c):
            base = c * CP
            pltpu.sync_copy(h_h.at[pl.ds(base, CP)], rows_v)
            pltpu.sync_copy(hier_h.at[pl.ds(base, CP)], hidx_v)
            pltpu.sync_copy(rows_v, pool_s.at[hidx_v], add=True)

        plsc.subcore_barrier()
        pltpu.sync_copy(pool_s.at[pl.ds(sid * MPT, MPT)],
                        out_h.at[cid, pl.ds(sid * MPT, MPT)])

    return k(h2, hier, zerosD)


def _tc_first(x, Wi, bi, Wrel, Wself, b):
    BN = 512
    G = N_AL // BN

    def body(x_r, wi_r, bi_r, wr_r, ws_r, b_r, hr_r, st_r):
        h = jnp.maximum(
            jnp.dot(x_r[...], wi_r[...], preferred_element_type=f32) + bi_r[...],
            0.0)
        for r in range(R):
            hr_r[r] = jnp.dot(h, wr_r[r], preferred_element_type=f32)
        st_r[...] = jnp.dot(h, ws_r[...], preferred_element_type=f32) + b_r[...]

    return pl.pallas_call(
        body,
        grid=(G,),
        in_specs=[
            pl.BlockSpec((BN, D), lambda i: (i, 0)),
            pl.BlockSpec((D, D), lambda i: (0, 0)),
            pl.BlockSpec((1, D), lambda i: (0, 0)),
            pl.BlockSpec((R, D, D), lambda i: (0, 0, 0)),
            pl.BlockSpec((D, D), lambda i: (0, 0)),
            pl.BlockSpec((1, D), lambda i: (0, 0)),
        ],
        out_specs=[
            pl.BlockSpec((R, BN, D), lambda i: (0, i, 0)),
            pl.BlockSpec((BN, D), lambda i: (i, 0)),
        ],
        out_shape=[
            jax.ShapeDtypeStruct((R, N_AL, D), f32),
            jax.ShapeDtypeStruct((N_AL, D), f32),
        ],
    )(x, Wi, bi, Wrel, Wself, b)


def _tc_mid(aggP, degP, st, Wrel, Wself, b, NN, BN):
    G = NN // BN

    def body(ap_r, dp_r, st_r, wr_r, ws_r, b_r, hr_r, st_o):
        agg = ap_r[0] + ap_r[1]
        deg = jnp.maximum(jnp.sum(dp_r[...], axis=(0, 1)), 1.0)
        h = jnp.maximum(agg / deg[:, None] + st_r[...], 0.0)
        for r in range(R):
            hr_r[r] = jnp.dot(h, wr_r[r], preferred_element_type=f32)
        st_o[...] = jnp.dot(h, ws_r[...], preferred_element_type=f32) + b_r[...]

    return pl.pallas_call(
        body,
        grid=(G,),
        in_specs=[
            pl.BlockSpec((NC, BN, D), lambda i: (0, i, 0)),
            pl.BlockSpec((NC, NS, BN), lambda i: (0, 0, i)),
            pl.BlockSpec((BN, D), lambda i: (i, 0)),
            pl.BlockSpec((R, D, D), lambda i: (0, 0, 0)),
            pl.BlockSpec((D, D), lambda i: (0, 0)),
            pl.BlockSpec((1, D), lambda i: (0, 0)),
        ],
        out_specs=[
            pl.BlockSpec((R, BN, D), lambda i: (0, i, 0)),
            pl.BlockSpec((BN, D), lambda i: (i, 0)),
        ],
        out_shape=[
            jax.ShapeDtypeStruct((R, NN, D), f32),
            jax.ShapeDtypeStruct((NN, D), f32),
        ],
    )(aggP, degP, st, Wrel, Wself, b)


def _tc_h(aggP, degP, st, NN, BN):
    G = NN // BN

    def body(ap_r, dp_r, st_r, h_o):
        agg = ap_r[0] + ap_r[1]
        deg = jnp.maximum(jnp.sum(dp_r[...], axis=(0, 1)), 1.0)
        h_o[...] = jnp.maximum(agg / deg[:, None] + st_r[...], 0.0)

    return pl.pallas_call(
        body,
        grid=(G,),
        in_specs=[
            pl.BlockSpec((NC, BN, D), lambda i: (0, i, 0)),
            pl.BlockSpec((NC, NS, BN), lambda i: (0, 0, i)),
            pl.BlockSpec((BN, D), lambda i: (i, 0)),
        ],
        out_specs=pl.BlockSpec((BN, D), lambda i: (i, 0)),
        out_shape=jax.ShapeDtypeStruct((NN, D), f32),
    )(aggP, degP, st)


def _tc_pool_mlp(poolP, cntP, Wrel, Wself, b):
    BN = 256
    G = M // BN

    def body(pp_r, cp_r, wr_r, ws_r, b_r, pr_r, st_o):
        cnt = jnp.maximum(jnp.sum(cp_r[...], axis=(0, 1)), 1.0)
        pooled = (pp_r[0] + pp_r[1]) / cnt[:, None]
        for r in range(R):
            pr_r[r] = jnp.dot(pooled, wr_r[r], preferred_element_type=f32)
        st_o[...] = jnp.dot(pooled, ws_r[...], preferred_element_type=f32) + b_r[...]

    return pl.pallas_call(
        body,
        grid=(G,),
        in_specs=[
            pl.BlockSpec((NC, BN, D), lambda i: (0, i, 0)),
            pl.BlockSpec((NC, NS, BN), lambda i: (0, 0, i)),
            pl.BlockSpec((R, D, D), lambda i: (0, 0, 0)),
            pl.BlockSpec((D, D), lambda i: (0, 0)),
            pl.BlockSpec((1, D), lambda i: (0, 0)),
        ],
        out_specs=[
            pl.BlockSpec((R, BN, D), lambda i: (0, i, 0)),
            pl.BlockSpec((BN, D), lambda i: (i, 0)),
        ],
        out_shape=[
            jax.ShapeDtypeStruct((R, M, D), f32),
            jax.ShapeDtypeStruct((M, D), f32),
        ],
    )(poolP, cntP, Wrel, Wself, b)


def _tc_final(maggP, degmP, pst, Wf, bf):
    BN = 256
    G = M // BN

    def body(ap_r, dp_r, st_r, wf_r, bf_r, o_r):
        deg = jnp.maximum(jnp.sum(dp_r[...], axis=(0, 1)), 1.0)
        p2 = jnp.maximum((ap_r[0] + ap_r[1]) / deg[:, None] + st_r[...], 0.0)
        o_r[...] = jnp.maximum(
            jnp.dot(p2, wf_r[...], preferred_element_type=f32) + bf_r[...], 0.0)

    return pl.pallas_call(
        body,
        grid=(G,),
        in_specs=[
            pl.BlockSpec((NC, BN, D), lambda i: (0, i, 0)),
            pl.BlockSpec((NC, NS, BN), lambda i: (0, 0, i)),
            pl.BlockSpec((BN, D), lambda i: (i, 0)),
            pl.BlockSpec((D, D), lambda i: (0, 0)),
            pl.BlockSpec((1, D), lambda i: (0, 0)),
        ],
        out_specs=pl.BlockSpec((BN, D), lambda i: (i, 0)),
        out_shape=jax.ShapeDtypeStruct((M, D), f32),
    )(maggP, degmP, pst, Wf, bf)


def kernel(x, edge_index, edge_type, hierarchy,
           W_init, b_init, W_rel_bu, W_self_bu, b_bu,
           W_rel_mod, W_self_mod, b_mod, W_fin, b_fin):
    src = edge_index[0]
    dst = edge_index[1]
    zerosD = jnp.zeros((CE, D), f32)

    ridx, mridx, mdst = _sc_idx(src, dst, edge_type, hierarchy)

    def _tok_i32(v):
        return jnp.minimum(v.astype(i32), 0)

    degP = _sc_hist(dst + _tok_i32(ridx[0]), N_AL, E, CE)
    degmP = _sc_hist(mdst + _tok_i32(degP[0, 0, 0]), M, E, CE)
    cntP = _sc_hist(hierarchy + _tok_i32(degmP[0, 0, 0]), M, N, CP)
    ridx = ridx + _tok_i32(cntP[0, 0, 0])

    x_pad = jnp.concatenate([x, jnp.zeros((N_AL - N, D), f32)], axis=0)
    hr0, st0 = _tc_first(x_pad, W_init, b_init.reshape(1, D),
                         W_rel_bu[0], W_self_bu[0], b_bu[0].reshape(1, D))
    aggP0 = _sc_edge_agg(hr0.reshape(R * N_AL, D), ridx, dst, N_AL, zerosD, 2)
    hr1, st1 = _tc_mid(aggP0, degP, st0,
                       W_rel_bu[1], W_self_bu[1], b_bu[1].reshape(1, D),
                       N_AL, 512)
    aggP1 = _sc_edge_agg(hr1.reshape(R * N_AL, D), ridx, dst, N_AL, zerosD, 2)
    h2 = _tc_h(aggP1, degP, st1, N_AL, 512)

    poolP = _sc_pool(h2, hierarchy, zerosD)
    pr0, pst0 = _tc_pool_mlp(poolP, cntP,
                             W_rel_mod[0], W_self_mod[0], b_mod[0].reshape(1, D))
    maggP0 = _sc_edge_agg(pr0.reshape(R * M, D), mridx, mdst, M, zerosD, 2)
    pr1, pst1 = _tc_mid(maggP0, degmP, pst0,
                        W_rel_mod[1], W_self_mod[1], b_mod[1].reshape(1, D),
                        M, 256)
    maggP1 = _sc_edge_agg(pr1.reshape(R * M, D), mridx, mdst, M, zerosD, 2)
    return _tc_final(maggP1, degmP, pst1, W_fin, b_fin.reshape(1, D))
